# baseline (device time: 18164 ns/iter reference)
import jax
import jax.numpy as jnp
from jax import lax
from jax.experimental import pallas as pl
from jax.experimental.pallas import tpu as pltpu

N_DEV = 32


def kernel(x):
    m_per, n = x.shape

    def signal_body(tok_ref):
        my_pos = lax.axis_index("i")
        barrier_sem = pltpu.get_barrier_semaphore()
        for o in range(1, N_DEV):
            pl.semaphore_signal(
                barrier_sem,
                inc=1,
                device_id=((my_pos + o) % N_DEV,),
                device_id_type=pl.DeviceIdType.MESH,
            )
        tok_ref[:, :] = jnp.zeros_like(tok_ref)

    tok = pl.pallas_call(
        signal_body,
        out_shape=jax.ShapeDtypeStruct((8, 128), jnp.float32),
        out_specs=pl.BlockSpec(memory_space=pltpu.VMEM),
        compiler_params=pltpu.CompilerParams(collective_id=0),
    )()

    def body(x_ref, tok_ref, out_ref, recv_ref, send_sems, recv_sems):
        my_pos = lax.axis_index("i")
        barrier_sem = pltpu.get_barrier_semaphore()

        xv = x_ref[:, :]
        val = jnp.max(xv, axis=0)
        rows = lax.broadcasted_iota(jnp.int32, (m_per, n), 0)
        masked = jnp.where(xv == val[None, :], rows, m_per)
        idx = jnp.min(masked, axis=0) + my_pos * m_per
        recv_ref[my_pos, 0, :] = val
        recv_ref[my_pos, 1, :] = idx.astype(jnp.float32)

        pl.semaphore_wait(barrier_sem, N_DEV - 1)

        sends = []
        for o in range(1, N_DEV):
            peer = (my_pos - o) % N_DEV
            rdma = pltpu.make_async_remote_copy(
                src_ref=recv_ref.at[my_pos],
                dst_ref=recv_ref.at[my_pos],
                send_sem=send_sems.at[o],
                recv_sem=recv_sems.at[my_pos],
                device_id=(peer,),
                device_id_type=pl.DeviceIdType.MESH,
            )
            rdma.start()
            sends.append(rdma)

        for p in range(1, N_DEV):
            src = my_pos ^ p
            recv = pltpu.make_async_remote_copy(
                src_ref=recv_ref.at[src],
                dst_ref=recv_ref.at[src],
                send_sem=send_sems.at[p],
                recv_sem=recv_sems.at[src],
                device_id=(src,),
                device_id_type=pl.DeviceIdType.MESH,
            )
            recv.wait_recv()

        vals = recv_ref[:, 0, :]
        idxs = recv_ref[:, 1, :]
        m = jnp.max(vals, axis=0)
        gi = jnp.min(
            jnp.where(vals == m[None, :], idxs, float(N_DEV * m_per)), axis=0
        )
        out_ref[0, :] = m
        out_ref[1, :] = gi

        for rdma in sends:
            rdma.wait_send()

    return pl.pallas_call(
        body,
        out_shape=jax.ShapeDtypeStruct((2, n), jnp.float32),
        in_specs=[
            pl.BlockSpec(memory_space=pltpu.VMEM),
            pl.BlockSpec(memory_space=pltpu.VMEM),
        ],
        out_specs=pl.BlockSpec(memory_space=pltpu.VMEM),
        scratch_shapes=[
            pltpu.VMEM((N_DEV, 2, n), jnp.float32),
            pltpu.SemaphoreType.DMA((N_DEV,)),
            pltpu.SemaphoreType.DMA((N_DEV,)),
        ],
        compiler_params=pltpu.CompilerParams(collective_id=0),
    )(x, tok)


# device time: 18137 ns/iter; 1.0015x vs baseline; 1.0015x over previous
import jax
import jax.numpy as jnp
from jax import lax
from jax.experimental import pallas as pl
from jax.experimental.pallas import tpu as pltpu

N_DEV = 32


def kernel(x):
    m_per, n = x.shape

    def body(x_ref, out_ref, recv_ref, send_sems, recv_sems):
        my_pos = lax.axis_index("i")

        barrier_sem = pltpu.get_barrier_semaphore()
        for o in range(1, N_DEV):
            pl.semaphore_signal(
                barrier_sem,
                inc=1,
                device_id=((my_pos + o) % N_DEV,),
                device_id_type=pl.DeviceIdType.MESH,
            )

        xv = x_ref[:, :]
        val = jnp.max(xv, axis=0)
        rows = lax.broadcasted_iota(jnp.int32, (m_per, n), 0)
        masked = jnp.where(xv == val[None, :], rows, m_per)
        idx = jnp.min(masked, axis=0) + my_pos * m_per
        recv_ref[my_pos, 0, :] = val
        recv_ref[my_pos, 1, :] = idx.astype(jnp.float32)

        pl.semaphore_wait(barrier_sem, N_DEV - 1)

        sends = []
        for o in range(1, N_DEV):
            peer = (my_pos - o) % N_DEV
            rdma = pltpu.make_async_remote_copy(
                src_ref=recv_ref.at[my_pos],
                dst_ref=recv_ref.at[my_pos],
                send_sem=send_sems.at[o],
                recv_sem=recv_sems.at[my_pos],
                device_id=(peer,),
                device_id_type=pl.DeviceIdType.MESH,
            )
            rdma.start()
            sends.append(rdma)

        for p in range(1, N_DEV):
            src = my_pos ^ p
            recv = pltpu.make_async_remote_copy(
                src_ref=recv_ref.at[src],
                dst_ref=recv_ref.at[src],
                send_sem=send_sems.at[p],
                recv_sem=recv_sems.at[src],
                device_id=(src,),
                device_id_type=pl.DeviceIdType.MESH,
            )
            recv.wait_recv()

        vals = recv_ref[:, 0, :]
        idxs = recv_ref[:, 1, :]
        m = jnp.max(vals, axis=0)
        gi = jnp.min(
            jnp.where(vals == m[None, :], idxs, float(N_DEV * m_per)), axis=0
        )
        out_ref[0, :] = m
        out_ref[1, :] = gi

        for rdma in sends:
            rdma.wait_send()

    return pl.pallas_call(
        body,
        out_shape=jax.ShapeDtypeStruct((2, n), jnp.float32),
        in_specs=[pl.BlockSpec(memory_space=pltpu.VMEM)],
        out_specs=pl.BlockSpec(memory_space=pltpu.VMEM),
        scratch_shapes=[
            pltpu.VMEM((N_DEV, 2, n), jnp.float32),
            pltpu.SemaphoreType.DMA((N_DEV,)),
            pltpu.SemaphoreType.DMA((N_DEV,)),
        ],
        compiler_params=pltpu.CompilerParams(collective_id=0),
    )(x)
